# baseline (speedup 1.0000x reference)
import functools

import jax
import jax.numpy as jnp
from jax import lax
from jax.experimental import pallas as pl
from jax.experimental.pallas import tpu as pltpu
from jax.experimental.pallas import tpu_sc as plsc

NC = 2
NS = 16
L = 16
NW = NC * NS

B = 4096
S = 50
D = 64
CH = 128
GB = D // 8
NBC = B // CH
NQ = S * GB * NBC
SUB = CH // L
NBUF = 5

_mesh = plsc.VectorSubcoreMesh(
    core_axis_name="c", subcore_axis_name="s", num_cores=NC, num_subcores=NS
)


@functools.partial(
    pl.kernel,
    mesh=_mesh,
    compiler_params=pltpu.CompilerParams(
        needs_layout_passes=False, use_tc_tiling_on_sc=False
    ),
    out_type=(
        jax.ShapeDtypeStruct((NQ, 8, CH), jnp.float32),
        jax.ShapeDtypeStruct((NQ, 8, CH), jnp.float32),
    ),
    scratch_types=[
        pltpu.VMEM((S, CH), jnp.int32),
        pltpu.VMEM((NBUF, CH, D), jnp.float32),
        pltpu.VMEM((NBUF, GB, 8, CH), jnp.float32),
        pltpu.SemaphoreType.DMA,
        pltpu.SemaphoreType.DMA,
    ],
)
def _emb_lookup(idx_hbm, table_hbm, fwd_hbm, bwd_hbm,
                idx_v, bufs, tbufs, gsem, wsem):
    wid = lax.axis_index("s") * NC + lax.axis_index("c")

    pltpu.sync_copy(idx_hbm.at[:, pl.ds(wid * CH, CH)], idx_v)

    lanes = lax.iota(jnp.int32, L)
    zeros = jnp.zeros((L,), jnp.float32)

    for k in range(NBUF):
        pltpu.async_copy(table_hbm.at[idx_v.at[k]], bufs.at[k], gsem)

    def chunk_body(j, _):
        b = lax.rem(j, NBUF)
        tb = lax.rem(j, NBUF)

        pltpu.make_async_copy(
            table_hbm.at[idx_v.at[0]], bufs.at[b], gsem
        ).wait()

        @pl.when(j >= NBUF)
        def _drain():
            for _k in range(2):
                pltpu.make_async_copy(
                    tbufs.at[0], fwd_hbm.at[pl.ds(0, GB)], wsem
                ).wait()

        masks = [idx_v[j, pl.ds(c * L, L)] == 0 for c in range(SUB)]
        any_m = masks[0]
        for c in range(1, SUB):
            any_m = any_m | masks[c]
        has_pad = jnp.max(jnp.where(any_m, 1, 0))

        @pl.when(has_pad > 0)
        def _fix():
            def fix_body(c, _f):
                m = idx_v[j, pl.ds(c * L, L)] == 0
                rows = c * L + lanes
                for col in range(D):
                    cols = jnp.full((L,), col, jnp.int32)
                    plsc.store_scatter(bufs.at[b], [rows, cols], zeros,
                                       mask=m)
                return _f

            lax.fori_loop(0, SUB, fix_body, None)

        for d in range(D):
            cols = jnp.full((L,), d, jnp.int32)
            for t in range(SUB):
                v = plsc.load_gather(bufs.at[b], [t * L + lanes, cols])
                tbufs[tb, d // 8, d % 8, pl.ds(t * L, L)] = v

        jn = j + NBUF

        @pl.when(jn < S)
        def _next_gather():
            pltpu.async_copy(
                table_hbm.at[idx_v.at[jn]], bufs.at[lax.rem(jn, NBUF)],
                gsem,
            )

        for gg in range(GB):
            piece = tbufs.at[tb, gg]
            pltpu.async_copy(
                piece, fwd_hbm.at[(j * GB + gg) * NBC + wid], wsem
            )
            pltpu.async_copy(
                piece,
                bwd_hbm.at[((S - 1 - j) * GB + gg) * NBC + wid],
                wsem,
            )
        return _

    lax.fori_loop(0, S, chunk_body, None)

    for _ in range(2 * NBUF):
        pltpu.make_async_copy(
            tbufs.at[0], fwd_hbm.at[pl.ds(0, GB)], wsem
        ).wait()


def kernel(sentence_index, embedding):
    idx_t = sentence_index.astype(jnp.int32).T
    fwd5, bwd5 = _emb_lookup(idx_t, embedding)

    def to_native(o5):
        return (
            o5.reshape(S, GB, NBC, 8, CH)
            .transpose(2, 4, 0, 1, 3)
            .reshape(B, S, D)
        )

    return to_native(fwd5), to_native(bwd5)


# speedup vs baseline: 1.2306x; 1.2306x over previous
import functools

import jax
import jax.numpy as jnp
from jax import lax
from jax.experimental import pallas as pl
from jax.experimental.pallas import tpu as pltpu
from jax.experimental.pallas import tpu_sc as plsc

NC = 2
NS = 16
L = 16
NW = NC * NS

B = 4096
S = 50
D = 64
CH = 128
GB = D // 8
NBC = B // CH
NQ = S * GB * NBC
SUB = CH // L
NBUF = 5

_mesh = plsc.VectorSubcoreMesh(
    core_axis_name="c", subcore_axis_name="s", num_cores=NC, num_subcores=NS
)


@functools.partial(
    pl.kernel,
    mesh=_mesh,
    compiler_params=pltpu.CompilerParams(
        needs_layout_passes=False, use_tc_tiling_on_sc=False
    ),
    out_type=(
        jax.ShapeDtypeStruct((NQ, 8, CH), jnp.float32),
        jax.ShapeDtypeStruct((NQ, 8, CH), jnp.float32),
    ),
    scratch_types=[
        pltpu.VMEM((S, CH), jnp.int32),
        pltpu.VMEM((NBUF, CH, D), jnp.float32),
        pltpu.VMEM((NBUF, D, CH), jnp.float32),
        pltpu.SemaphoreType.DMA,
        pltpu.SemaphoreType.DMA,
    ],
)
def _emb_lookup(idx_hbm, table_hbm, fwd_hbm, bwd_hbm,
                idx_v, bufs, tbufs, gsem, wsem):
    wid = lax.axis_index("s") * NC + lax.axis_index("c")

    pltpu.sync_copy(idx_hbm.at[:, pl.ds(wid * CH, CH)], idx_v)

    lanes = lax.iota(jnp.int32, L)
    zeros = jnp.zeros((L,), jnp.float32)
    rot = [lax.rem(lanes + k, L) for k in range(L)]

    for k in range(NBUF):
        pltpu.async_copy(table_hbm.at[idx_v.at[k]], bufs.at[k], gsem)

    def chunk_body(j, _):
        b = lax.rem(j, NBUF)
        tb = lax.rem(j, NBUF)

        pltpu.make_async_copy(
            table_hbm.at[idx_v.at[0]], bufs.at[b], gsem
        ).wait()

        @pl.when(j >= NBUF)
        def _drain():
            for _k in range(2):
                pltpu.make_async_copy(
                    table_hbm.at[pl.ds(0, CH)], bufs.at[0], wsem
                ).wait()

        masks = [idx_v[j, pl.ds(c * L, L)] == 0 for c in range(SUB)]
        any_m = masks[0]
        for c in range(1, SUB):
            any_m = any_m | masks[c]
        has_pad = jnp.max(jnp.where(any_m, 1, 0))

        @pl.when(has_pad > 0)
        def _fix():
            def fix_body(c, _f):
                m = idx_v[j, pl.ds(c * L, L)] == 0
                rows = c * L + lanes
                for col in range(D):
                    cols = jnp.full((L,), col, jnp.int32)
                    plsc.store_scatter(bufs.at[b], [rows, cols], zeros,
                                       mask=m)
                return _f

            lax.fori_loop(0, SUB, fix_body, None)

        tdst = tbufs.at[tb]
        tsrc = bufs.at[b]

        def tr_body(t8, _t):
            toks = t8 * L + lanes
            for k in range(L):
                rk = rot[k]
                for d0 in range(0, D, L):
                    cols = d0 + rk
                    v = plsc.load_gather(tsrc, [toks, cols])
                    plsc.store_scatter(tdst, [cols, toks], v)
            return _t

        lax.fori_loop(0, SUB, tr_body, None)

        jn = j + NBUF

        @pl.when(jn < S)
        def _next_gather():
            pltpu.async_copy(
                table_hbm.at[idx_v.at[jn]], bufs.at[lax.rem(jn, NBUF)],
                gsem,
            )

        for gg in range(GB):
            piece = tbufs.at[tb, pl.ds(8 * gg, 8)]
            pltpu.async_copy(
                piece, fwd_hbm.at[(j * GB + gg) * NBC + wid], wsem
            )
            pltpu.async_copy(
                piece,
                bwd_hbm.at[((S - 1 - j) * GB + gg) * NBC + wid],
                wsem,
            )
        return _

    lax.fori_loop(0, S, chunk_body, None)

    for _ in range(2 * NBUF):
        pltpu.make_async_copy(
            table_hbm.at[pl.ds(0, CH)], bufs.at[0], wsem
        ).wait()


def kernel(sentence_index, embedding):
    idx_t = sentence_index.astype(jnp.int32).T
    fwd5, bwd5 = _emb_lookup(idx_t, embedding)

    def to_native(o5):
        return (
            o5.reshape(S, GB, NBC, 8, CH)
            .transpose(2, 4, 0, 1, 3)
            .reshape(B, S, D)
        )

    return to_native(fwd5), to_native(bwd5)


# speedup vs baseline: 1.2788x; 1.0392x over previous
import functools

import jax
import jax.numpy as jnp
from jax import lax
from jax.experimental import pallas as pl
from jax.experimental.pallas import tpu as pltpu
from jax.experimental.pallas import tpu_sc as plsc

NC = 2
NS = 16
L = 16
NW = NC * NS

B = 4096
S = 50
D = 64
PW = 2 * D
CH = 128
GB = D // 8
NBC = B // CH
NQ = S * GB * NBC
SUB = CH // L
NBUF = 4

_mesh = plsc.VectorSubcoreMesh(
    core_axis_name="c", subcore_axis_name="s", num_cores=NC, num_subcores=NS
)


@functools.partial(
    pl.kernel,
    mesh=_mesh,
    compiler_params=pltpu.CompilerParams(
        needs_layout_passes=False, use_tc_tiling_on_sc=True
    ),
    out_type=(
        jax.ShapeDtypeStruct((NQ, 8, CH), jnp.float32),
        jax.ShapeDtypeStruct((NQ, 8, CH), jnp.float32),
    ),
    scratch_types=[
        pltpu.VMEM((56, CH), jnp.int32),
        pltpu.VMEM((56, CH), jnp.int32),
        pltpu.VMEM((56, CH), jnp.int32),
        pltpu.VMEM((NBUF, CH, PW), jnp.float32),
        pltpu.VMEM((NBUF, D, CH), jnp.float32),
        pltpu.SemaphoreType.DMA,
        pltpu.SemaphoreType.DMA,
    ],
)
def _emb_lookup(idx_hbm, table_hbm, fwd_hbm, bwd_hbm,
                idx_v, pidx_v, off_v, bufs, tbufs, gsem, wsem):
    wid = lax.axis_index("s") * NC + lax.axis_index("c")

    pltpu.sync_copy(idx_hbm.at[:, pl.ds(wid * CH, CH)], idx_v.at[pl.ds(0, S)])

    lanes = lax.iota(jnp.int32, L)
    zeros = jnp.zeros((L,), jnp.float32)
    rot = [lax.rem(lanes + k, L) for k in range(L)]

    def pidx_body(j, _):
        for c in range(SUB):
            iv = idx_v[j, pl.ds(c * L, L)]
            pidx_v[j, pl.ds(c * L, L)] = lax.shift_right_logical(iv, 1)
            off_v[j, pl.ds(c * L, L)] = lax.shift_left(
                lax.bitwise_and(iv, 1), 6
            )
        return _

    lax.fori_loop(0, S, pidx_body, None)

    for k in range(NBUF):
        pltpu.async_copy(table_hbm.at[pidx_v.at[k]], bufs.at[k], gsem)

    def chunk_body(j, _):
        b = lax.rem(j, NBUF)

        pltpu.make_async_copy(
            table_hbm.at[pl.ds(0, CH)], bufs.at[b], gsem
        ).wait()

        @pl.when(j >= NBUF)
        def _drain():
            for _k in range(2):
                pltpu.make_async_copy(
                    table_hbm.at[pl.ds(0, D)], tbufs.at[0], wsem
                ).wait()

        masks = [idx_v[j, pl.ds(c * L, L)] == 0 for c in range(SUB)]
        any_m = masks[0]
        for c in range(1, SUB):
            any_m = any_m | masks[c]
        has_pad = jnp.max(jnp.where(any_m, 1, 0))

        @pl.when(has_pad > 0)
        def _fix():
            def fix_body(c, _f):
                m = idx_v[j, pl.ds(c * L, L)] == 0
                rows = c * L + lanes
                for col in range(PW):
                    cols = jnp.full((L,), col, jnp.int32)
                    plsc.store_scatter(bufs.at[b], [rows, cols], zeros,
                                       mask=m)
                return _f

            lax.fori_loop(0, SUB, fix_body, None)

        tdst = tbufs.at[b]
        tsrc = bufs.at[b]

        def tr_body(t8, _t):
            toks = t8 * L + lanes
            offs = off_v[j, pl.ds(t8 * L, L)]
            for k in range(L):
                rk = rot[k]
                for d0 in range(0, D, L):
                    cols = d0 + rk
                    v = plsc.load_gather(tsrc, [toks, offs + cols])
                    plsc.store_scatter(tdst, [cols, toks], v)
            return _t

        lax.fori_loop(0, SUB, tr_body, None)

        jn = j + NBUF

        @pl.when(jn < S)
        def _next_gather():
            pltpu.async_copy(
                table_hbm.at[pidx_v.at[jn]], bufs.at[lax.rem(jn, NBUF)],
                gsem,
            )

        for gg in range(GB):
            piece = tbufs.at[b, pl.ds(8 * gg, 8)]
            pltpu.async_copy(
                piece, fwd_hbm.at[(j * GB + gg) * NBC + wid], wsem
            )
            pltpu.async_copy(
                piece,
                bwd_hbm.at[((S - 1 - j) * GB + gg) * NBC + wid],
                wsem,
            )
        return _

    lax.fori_loop(0, S, chunk_body, None)

    for _ in range(2 * NBUF):
        pltpu.make_async_copy(
            table_hbm.at[pl.ds(0, D)], tbufs.at[0], wsem
        ).wait()


def kernel(sentence_index, embedding):
    idx_t = sentence_index.astype(jnp.int32).T
    table2 = embedding.reshape(500000, PW)
    fwd5, bwd5 = _emb_lookup(idx_t, table2)

    def to_native(o5):
        return (
            o5.reshape(S, GB, NBC, 8, CH)
            .transpose(2, 4, 0, 1, 3)
            .reshape(B, S, D)
        )

    return to_native(fwd5), to_native(bwd5)
